# initial kernel scaffold (unmeasured)
import jax
import jax.numpy as jnp
from jax import lax
from jax.experimental import pallas as pl
from jax.experimental.pallas import tpu as pltpu

N_DEV = 8
M, K_SHARD, N = 4096, 512, 8192
CH = M // N_DEV


def _gemm(x, w):
    BM, BN = 512, 2048

    def body(x_ref, w_ref, o_ref):
        o_ref[...] = jnp.dot(
            x_ref[...], w_ref[...], preferred_element_type=jnp.float32
        )

    return pl.pallas_call(
        body,
        grid=(N // BN, M // BM),
        in_specs=[
            pl.BlockSpec((BM, K_SHARD), lambda j, i: (i, 0)),
            pl.BlockSpec((K_SHARD, BN), lambda j, i: (0, j)),
        ],
        out_specs=pl.BlockSpec((BM, BN), lambda j, i: (i, j)),
        out_shape=jax.ShapeDtypeStruct((M, N), jnp.float32),
    )(x, w)


def _ring_allreduce(partial):

    def body(p_ref, o_ref, comm_ref, tmp_ref, send_sem, recv_sem,
             local_sem, store_sem, credit_sem):
        me = lax.axis_index("i")
        left = (me + N_DEV - 1) % N_DEV
        right = (me + 1) % N_DEV

        barrier_sem = pltpu.get_barrier_semaphore()
        for nbr in (left, right):
            pl.semaphore_signal(
                barrier_sem, inc=1,
                device_id=(nbr,), device_id_type=pl.DeviceIdType.MESH,
            )
        pl.semaphore_wait(barrier_sem, 2)

        def load_chunk(c, dst):
            cp = pltpu.make_async_copy(
                p_ref.at[pl.ds(c * CH, CH), :], dst, local_sem
            )
            cp.start()
            return cp

        def store_chunk(src, c):
            cp = pltpu.make_async_copy(
                src, o_ref.at[pl.ds(c * CH, CH), :], store_sem
            )
            cp.start()
            return cp

        load_chunk(me, comm_ref.at[0]).wait()
        for s in range(N_DEV - 1):
            if s > 0:
                pl.semaphore_wait(credit_sem, 1)
            rdma = pltpu.make_async_remote_copy(
                src_ref=comm_ref.at[0],
                dst_ref=comm_ref.at[1],
                send_sem=send_sem,
                recv_sem=recv_sem,
                device_id=(right,),
                device_id_type=pl.DeviceIdType.MESH,
            )
            rdma.start()
            c_recv = (me - s - 1) % N_DEV
            lcp = load_chunk(c_recv, tmp_ref)
            rdma.wait()
            lcp.wait()
            comm_ref[0] = comm_ref[1] + tmp_ref[...]
            pl.semaphore_signal(
                credit_sem, inc=1,
                device_id=(left,), device_id_type=pl.DeviceIdType.MESH,
            )

        own = (me + 1) % N_DEV
        store_chunk(comm_ref.at[0], own).wait()

        for t in range(N_DEV - 1):
            s_slot = t % 2
            r_slot = (t + 1) % 2
            pl.semaphore_wait(credit_sem, 1)
            rdma = pltpu.make_async_remote_copy(
                src_ref=comm_ref.at[s_slot],
                dst_ref=comm_ref.at[r_slot],
                send_sem=send_sem,
                recv_sem=recv_sem,
                device_id=(right,),
                device_id_type=pl.DeviceIdType.MESH,
            )
            rdma.start()
            rdma.wait()
            c_recv = (me - t) % N_DEV
            store_chunk(comm_ref.at[r_slot], c_recv).wait()
            if t < N_DEV - 2:
                pl.semaphore_signal(
                    credit_sem, inc=1,
                    device_id=(left,), device_id_type=pl.DeviceIdType.MESH,
                )

    return pl.pallas_call(
        body,
        out_shape=jax.ShapeDtypeStruct((M, N), jnp.float32),
        in_specs=[pl.BlockSpec(memory_space=pltpu.ANY)],
        out_specs=pl.BlockSpec(memory_space=pltpu.ANY),
        scratch_shapes=[
            pltpu.VMEM((2, CH, N), jnp.float32),
            pltpu.VMEM((CH, N), jnp.float32),
            pltpu.SemaphoreType.DMA,
            pltpu.SemaphoreType.DMA,
            pltpu.SemaphoreType.DMA,
            pltpu.SemaphoreType.DMA,
            pltpu.SemaphoreType.REGULAR,
        ],
        compiler_params=pltpu.CompilerParams(collective_id=0),
    )(partial)


def kernel(x, w_mat):
    partial = _gemm(x, w_mat)
    y = _ring_allreduce(partial)
    amax = jnp.max(jnp.abs(y))
    scale = amax / 448.0
    q = (y / scale).astype(jnp.float8_e4m3fn)
    return q.astype(jnp.float32) * scale


# baseline (device time: 3007843 ns/iter reference)
import jax
import jax.numpy as jnp
from jax import lax
from jax.experimental import pallas as pl
from jax.experimental.pallas import tpu as pltpu

N_DEV = 8
M, K_SHARD, N = 4096, 512, 8192
CH = M // N_DEV


def _gemm(x, w):
    BM, BN = 512, 2048

    def body(x_ref, w_ref, o_ref):
        o_ref[...] = jnp.dot(
            x_ref[...], w_ref[...],
            preferred_element_type=jnp.float32,
            precision=lax.Precision.HIGHEST,
        )

    return pl.pallas_call(
        body,
        grid=(N // BN, M // BM),
        in_specs=[
            pl.BlockSpec((BM, K_SHARD), lambda j, i: (i, 0)),
            pl.BlockSpec((K_SHARD, BN), lambda j, i: (0, j)),
        ],
        out_specs=pl.BlockSpec((BM, BN), lambda j, i: (i, j)),
        out_shape=jax.ShapeDtypeStruct((M, N), jnp.float32),
    )(x, w)


def _ring_allreduce(partial):

    def body(p_ref, o_ref, comm_ref, tmp_ref, send_sem, recv_sem,
             local_sem, store_sem, credit_sem):
        me = lax.axis_index("i")
        left = (me + N_DEV - 1) % N_DEV
        right = (me + 1) % N_DEV

        barrier_sem = pltpu.get_barrier_semaphore()
        for nbr in (left, right):
            pl.semaphore_signal(
                barrier_sem, inc=1,
                device_id=(nbr,), device_id_type=pl.DeviceIdType.MESH,
            )
        pl.semaphore_wait(barrier_sem, 2)

        def load_chunk(c, dst):
            cp = pltpu.make_async_copy(
                p_ref.at[pl.ds(c * CH, CH), :], dst, local_sem
            )
            cp.start()
            return cp

        def store_chunk(src, c):
            cp = pltpu.make_async_copy(
                src, o_ref.at[pl.ds(c * CH, CH), :], store_sem
            )
            cp.start()
            return cp

        load_chunk(me, comm_ref.at[0]).wait()
        for s in range(N_DEV - 1):
            if s > 0:
                pl.semaphore_wait(credit_sem, 1)
            rdma = pltpu.make_async_remote_copy(
                src_ref=comm_ref.at[0],
                dst_ref=comm_ref.at[1],
                send_sem=send_sem,
                recv_sem=recv_sem,
                device_id=(right,),
                device_id_type=pl.DeviceIdType.MESH,
            )
            rdma.start()
            c_recv = (me - s - 1) % N_DEV
            lcp = load_chunk(c_recv, tmp_ref)
            rdma.wait()
            lcp.wait()
            comm_ref[0] = comm_ref[1] + tmp_ref[...]
            pl.semaphore_signal(
                credit_sem, inc=1,
                device_id=(left,), device_id_type=pl.DeviceIdType.MESH,
            )

        own = (me + 1) % N_DEV
        store_chunk(comm_ref.at[0], own).wait()

        for t in range(N_DEV - 1):
            s_slot = t % 2
            r_slot = (t + 1) % 2
            pl.semaphore_wait(credit_sem, 1)
            rdma = pltpu.make_async_remote_copy(
                src_ref=comm_ref.at[s_slot],
                dst_ref=comm_ref.at[r_slot],
                send_sem=send_sem,
                recv_sem=recv_sem,
                device_id=(right,),
                device_id_type=pl.DeviceIdType.MESH,
            )
            rdma.start()
            rdma.wait()
            c_recv = (me - t) % N_DEV
            store_chunk(comm_ref.at[r_slot], c_recv).wait()
            if t < N_DEV - 2:
                pl.semaphore_signal(
                    credit_sem, inc=1,
                    device_id=(left,), device_id_type=pl.DeviceIdType.MESH,
                )

    return pl.pallas_call(
        body,
        out_shape=jax.ShapeDtypeStruct((M, N), jnp.float32),
        in_specs=[pl.BlockSpec(memory_space=pl.ANY)],
        out_specs=pl.BlockSpec(memory_space=pl.ANY),
        scratch_shapes=[
            pltpu.VMEM((2, CH, N), jnp.float32),
            pltpu.VMEM((CH, N), jnp.float32),
            pltpu.SemaphoreType.DMA,
            pltpu.SemaphoreType.DMA,
            pltpu.SemaphoreType.DMA,
            pltpu.SemaphoreType.DMA,
            pltpu.SemaphoreType.REGULAR,
        ],
        compiler_params=pltpu.CompilerParams(
            collective_id=0, vmem_limit_bytes=100 * 1024 * 1024
        ),
    )(partial)


def kernel(x, w_mat):
    partial = _gemm(x, w_mat)
    y = _ring_allreduce(partial)
    amax = jnp.max(jnp.abs(y))
    scale = amax / 448.0
    q = (y / scale).astype(jnp.float8_e4m3fn)
    q = lax.optimization_barrier(q)
    return q.astype(jnp.float32) * scale


# device time: 1837384 ns/iter; 1.6370x vs baseline; 1.6370x over previous
import jax
import jax.numpy as jnp
from jax import lax
from jax.experimental import pallas as pl
from jax.experimental.pallas import tpu as pltpu

N_DEV = 8
M, K_SHARD, N = 4096, 512, 8192
CH = M // N_DEV
NSPLIT = 2
NH = N // NSPLIT


def _fused(x, w):
    def body(x_ref, w_ref, o_ref, comm_ref, q_ref, deq_ref, amx_ref,
             send_sem, recv_sem, amx_send, amx_recv, store_sem, load_sem,
             credit_sem):
        me = lax.axis_index("i")
        left = (me + N_DEV - 1) % N_DEV
        right = (me + 1) % N_DEV
        own = (me + 1) % N_DEV

        barrier_sem = pltpu.get_barrier_semaphore()
        for nbr in (left, right):
            pl.semaphore_signal(
                barrier_sem, inc=1,
                device_id=(nbr,), device_id_type=pl.DeviceIdType.MESH,
            )
        pl.semaphore_wait(barrier_sem, 2)

        def credit_wait():
            pl.semaphore_wait(credit_sem, 1)

        def credit_signal():
            pl.semaphore_signal(
                credit_sem, inc=1,
                device_id=(left,), device_id_type=pl.DeviceIdType.MESH,
            )

        def pc(c, h):
            return jnp.dot(
                x_ref[pl.ds(c * CH, CH), :],
                w_ref[:, pl.ds(h * NH, NH)],
                preferred_element_type=jnp.float32,
                precision=lax.Precision.HIGHEST,
            )

        def ring_rdma(src, dst):
            return pltpu.make_async_remote_copy(
                src_ref=src, dst_ref=dst,
                send_sem=send_sem, recv_sem=recv_sem,
                device_id=(right,), device_id_type=pl.DeviceIdType.MESH,
            )

        def out_at(c, h):
            return o_ref.at[pl.ds(c * CH, CH), pl.ds(h * NH, NH)]

        def store_chunk(src, c, h):
            cp = pltpu.make_async_copy(src, out_at(c, h), store_sem)
            cp.start()
            return cp

        def wait_prev_store(c, h):
            pltpu.make_async_copy(deq_ref, out_at(c, h), store_sem).wait()

        local_am = None
        for h in range(NSPLIT):
            comm_ref[0] = pc(me, h)

            def rs_step(s, _, h=h):
                s_slot = s % 2
                r_slot = (s + 1) % 2
                if h == 0:
                    @pl.when(s > 0)
                    def _():
                        credit_wait()
                else:
                    credit_wait()
                rdma = ring_rdma(comm_ref.at[s_slot], comm_ref.at[r_slot])
                rdma.start()
                deq_ref[...] = pc((me - s - 1) % N_DEV, h)
                rdma.wait()
                comm_ref[r_slot] = comm_ref[r_slot] + deq_ref[...]

                @pl.when(s < N_DEV - 2)
                def _():
                    credit_signal()
                return 0

            lax.fori_loop(0, N_DEV - 1, rs_step, 0)
            am_h = jnp.max(jnp.abs(comm_ref[1]))
            local_am = am_h if local_am is None else jnp.maximum(local_am, am_h)
            store_chunk(comm_ref.at[1], own, h).wait()
            credit_signal()

        amx_ref[me] = jnp.broadcast_to(local_am, (8, 128))
        sends = []
        for d in range(1, N_DEV):
            j = (me + d) % N_DEV
            r = pltpu.make_async_remote_copy(
                src_ref=amx_ref.at[me], dst_ref=amx_ref.at[me],
                send_sem=amx_send.at[d - 1], recv_sem=amx_recv.at[me],
                device_id=(j,), device_id_type=pl.DeviceIdType.MESH,
            )
            r.start()
            sends.append(r)
        for d in range(1, N_DEV):
            j = (me + d) % N_DEV
            pltpu.make_async_remote_copy(
                src_ref=amx_ref.at[j], dst_ref=amx_ref.at[j],
                send_sem=amx_send.at[d - 1], recv_sem=amx_recv.at[j],
                device_id=(j,), device_id_type=pl.DeviceIdType.MESH,
            ).wait_recv()
        for r in sends:
            r.wait_send()
        g = amx_ref[0]
        for j in range(1, N_DEV):
            g = jnp.maximum(g, amx_ref[j])
        g_am = jnp.max(g)
        scale = g_am / 448.0
        inv = 448.0 / g_am

        last_c = (me - (N_DEV - 2)) % N_DEV
        for h in range(NSPLIT):
            if h > 0:
                wait_prev_store(last_c, h - 1)
            lcp = pltpu.make_async_copy(out_at(own, h), deq_ref, load_sem)
            lcp.start()
            lcp.wait()
            q_ref[0] = (deq_ref[...] * inv).astype(jnp.float8_e4m3fn)
            deq_ref[...] = q_ref[0].astype(jnp.float32) * scale
            store_chunk(deq_ref, own, h).wait()

            def ag_step(t, _, h=h):
                s_slot = t % 2
                r_slot = (t + 1) % 2
                credit_wait()
                rdma = ring_rdma(q_ref.at[s_slot], q_ref.at[r_slot])
                rdma.start()
                rdma.wait()

                @pl.when(t > 0)
                def _():
                    wait_prev_store((me - t + 1) % N_DEV, h)
                deq_ref[...] = q_ref[r_slot].astype(jnp.float32) * scale
                if h == NSPLIT - 1:
                    @pl.when(t < N_DEV - 2)
                    def _():
                        credit_signal()
                else:
                    credit_signal()
                store_chunk(deq_ref, (me - t) % N_DEV, h)
                return 0

            lax.fori_loop(0, N_DEV - 1, ag_step, 0)
        wait_prev_store(last_c, NSPLIT - 1)

    return pl.pallas_call(
        body,
        out_shape=jax.ShapeDtypeStruct((M, N), jnp.float32),
        in_specs=[
            pl.BlockSpec(memory_space=pltpu.MemorySpace.VMEM),
            pl.BlockSpec(memory_space=pltpu.MemorySpace.VMEM),
        ],
        out_specs=pl.BlockSpec(memory_space=pl.ANY),
        scratch_shapes=[
            pltpu.VMEM((2, CH, NH), jnp.float32),
            pltpu.VMEM((2, CH, NH), jnp.float8_e4m3fn),
            pltpu.VMEM((CH, NH), jnp.float32),
            pltpu.VMEM((N_DEV, 8, 128), jnp.float32),
            pltpu.SemaphoreType.DMA,
            pltpu.SemaphoreType.DMA,
            pltpu.SemaphoreType.DMA((N_DEV - 1,)),
            pltpu.SemaphoreType.DMA((N_DEV,)),
            pltpu.SemaphoreType.DMA,
            pltpu.SemaphoreType.DMA,
            pltpu.SemaphoreType.REGULAR,
        ],
        compiler_params=pltpu.CompilerParams(
            collective_id=0, vmem_limit_bytes=63 * 1024 * 1024
        ),
    )(x, w)


def kernel(x, w_mat):
    return _fused(x, w_mat)


# device time: 1831035 ns/iter; 1.6427x vs baseline; 1.0035x over previous
import jax
import jax.numpy as jnp
from jax import lax
from jax.experimental import pallas as pl
from jax.experimental.pallas import tpu as pltpu

N_DEV = 8
M, K_SHARD, N = 4096, 512, 8192
CH = M // N_DEV
NSPLIT = 2
NH = N // NSPLIT


def _fused(x, w):
    def body(x_ref, w_ref, o_ref, comm_ref, q_ref, deq_ref, amx_ref,
             send_sem, recv_sem, amx_send, amx_recv, store_sem, load_sem,
             credit_sem):
        me = lax.axis_index("i")
        left = (me + N_DEV - 1) % N_DEV
        right = (me + 1) % N_DEV
        own = (me + 1) % N_DEV

        barrier_sem = pltpu.get_barrier_semaphore()
        for nbr in (left, right):
            pl.semaphore_signal(
                barrier_sem, inc=1,
                device_id=(nbr,), device_id_type=pl.DeviceIdType.MESH,
            )
        pl.semaphore_wait(barrier_sem, 2)

        def credit_wait():
            pl.semaphore_wait(credit_sem, 1)

        def credit_signal():
            pl.semaphore_signal(
                credit_sem, inc=1,
                device_id=(left,), device_id_type=pl.DeviceIdType.MESH,
            )

        def pc(c, h):
            return jnp.dot(
                x_ref[pl.ds(c * CH, CH), :],
                w_ref[:, pl.ds(h * NH, NH)],
                preferred_element_type=jnp.float32,
                precision=lax.Precision.HIGHEST,
            )

        def ring_rdma(src, dst):
            return pltpu.make_async_remote_copy(
                src_ref=src, dst_ref=dst,
                send_sem=send_sem, recv_sem=recv_sem,
                device_id=(right,), device_id_type=pl.DeviceIdType.MESH,
            )

        def out_at(c, h):
            return o_ref.at[pl.ds(c * CH, CH), pl.ds(h * NH, NH)]

        def store_chunk(src, c, h):
            cp = pltpu.make_async_copy(src, out_at(c, h), store_sem)
            cp.start()
            return cp

        def wait_prev_store(c, h):
            pltpu.make_async_copy(deq_ref, out_at(c, h), store_sem).wait()

        local_am = None
        for h in range(NSPLIT):
            comm_ref[0] = pc(me, h)

            def rs_step(s, _, h=h):
                s_slot = s % 2
                r_slot = (s + 1) % 2
                if h == 0:
                    @pl.when(s > 0)
                    def _():
                        credit_wait()
                else:
                    credit_wait()
                rdma = ring_rdma(comm_ref.at[s_slot], comm_ref.at[r_slot])
                rdma.start()
                deq_ref[...] = pc((me - s - 1) % N_DEV, h)
                rdma.wait()
                comm_ref[r_slot] = comm_ref[r_slot] + deq_ref[...]

                @pl.when(s < N_DEV - 2)
                def _():
                    credit_signal()
                return 0

            lax.fori_loop(0, N_DEV - 1, rs_step, 0)
            am_h = jnp.max(jnp.abs(comm_ref[1]))
            local_am = am_h if local_am is None else jnp.maximum(local_am, am_h)
            store_chunk(comm_ref.at[1], own, h).wait()
            credit_signal()

        amx_ref[me] = jnp.broadcast_to(local_am, (8, 128))
        sends = []
        for d in range(1, N_DEV):
            j = (me + d) % N_DEV
            r = pltpu.make_async_remote_copy(
                src_ref=amx_ref.at[me], dst_ref=amx_ref.at[me],
                send_sem=amx_send.at[d - 1], recv_sem=amx_recv.at[me],
                device_id=(j,), device_id_type=pl.DeviceIdType.MESH,
            )
            r.start()
            sends.append(r)
        for d in range(1, N_DEV):
            j = (me + d) % N_DEV
            pltpu.make_async_remote_copy(
                src_ref=amx_ref.at[j], dst_ref=amx_ref.at[j],
                send_sem=amx_send.at[d - 1], recv_sem=amx_recv.at[j],
                device_id=(j,), device_id_type=pl.DeviceIdType.MESH,
            ).wait_recv()
        for r in sends:
            r.wait_send()
        g = amx_ref[0]
        for j in range(1, N_DEV):
            g = jnp.maximum(g, amx_ref[j])
        g_am = jnp.max(g)
        scale = g_am / 448.0
        inv = 448.0 / g_am

        last_c = (me - (N_DEV - 2)) % N_DEV
        for h in range(NSPLIT):
            if h > 0:
                wait_prev_store(last_c, h - 1)
            lcp = pltpu.make_async_copy(out_at(own, h), deq_ref, load_sem)
            lcp.start()
            lcp.wait()
            q_ref[0] = (deq_ref[...] * inv).astype(jnp.float8_e4m3fn)
            deq_ref[...] = q_ref[0].astype(jnp.float32) * scale
            store_chunk(deq_ref, own, h).wait()

            def ag_step(t, _, h=h):
                s_slot = t % 2
                r_slot = (t + 1) % 2
                credit_wait()
                rdma = ring_rdma(q_ref.at[s_slot], q_ref.at[r_slot])
                rdma.start()

                @pl.when(t > 1)
                def _():
                    wait_prev_store((me - t + 2) % N_DEV, h)

                @pl.when(t > 0)
                def _():
                    deq_ref[...] = q_ref[s_slot].astype(jnp.float32) * scale
                    store_chunk(deq_ref, (me - (t - 1)) % N_DEV, h)
                rdma.wait()
                @pl.when(t < N_DEV - 2)
                def _():
                    credit_signal()
                return 0

            lax.fori_loop(0, N_DEV - 1, ag_step, 0)
            wait_prev_store(last_c, h)
            deq_ref[...] = q_ref[1].astype(jnp.float32) * scale
            store_chunk(deq_ref, last_c, h)
            if h < NSPLIT - 1:
                credit_signal()
        wait_prev_store(last_c, NSPLIT - 1)

    return pl.pallas_call(
        body,
        out_shape=jax.ShapeDtypeStruct((M, N), jnp.float32),
        in_specs=[
            pl.BlockSpec(memory_space=pltpu.MemorySpace.VMEM),
            pl.BlockSpec(memory_space=pltpu.MemorySpace.VMEM),
        ],
        out_specs=pl.BlockSpec(memory_space=pl.ANY),
        scratch_shapes=[
            pltpu.VMEM((2, CH, NH), jnp.float32),
            pltpu.VMEM((2, CH, NH), jnp.float8_e4m3fn),
            pltpu.VMEM((CH, NH), jnp.float32),
            pltpu.VMEM((N_DEV, 8, 128), jnp.float32),
            pltpu.SemaphoreType.DMA,
            pltpu.SemaphoreType.DMA,
            pltpu.SemaphoreType.DMA((N_DEV - 1,)),
            pltpu.SemaphoreType.DMA((N_DEV,)),
            pltpu.SemaphoreType.DMA,
            pltpu.SemaphoreType.DMA,
            pltpu.SemaphoreType.REGULAR,
        ],
        compiler_params=pltpu.CompilerParams(
            collective_id=0, vmem_limit_bytes=63 * 1024 * 1024
        ),
    )(x, w)


def kernel(x, w_mat):
    return _fused(x, w_mat)


# device time: 1045820 ns/iter; 2.8761x vs baseline; 1.7508x over previous
import jax
import jax.numpy as jnp
from jax import lax
from jax.experimental import pallas as pl
from jax.experimental.pallas import tpu as pltpu

N_DEV = 8
M, K_SHARD, N = 4096, 512, 8192
CH = M // N_DEV
NSPLIT = 2
NH = N // NSPLIT
NH2 = NH // 2


def _fused(x, w):
    def body(x_ref, w_ref, o_ref, comm_ref, q_ref, deq_ref, amx_ref,
             send_sem, recv_sem, amx_send, amx_recv, store_sem, load_sem,
             rs_credit, ag_credit):
        me = lax.axis_index("i")
        left = (me + N_DEV - 1) % N_DEV
        right = (me + 1) % N_DEV
        dst = (right, left)
        csrc = (left, right)
        own = ((me + 1) % N_DEV, (me + N_DEV - 1) % N_DEV)

        barrier_sem = pltpu.get_barrier_semaphore()
        for nbr in (left, right):
            pl.semaphore_signal(
                barrier_sem, inc=1,
                device_id=(nbr,), device_id_type=pl.DeviceIdType.MESH,
            )
        pl.semaphore_wait(barrier_sem, 2)

        def credit_wait(sem):
            for d in range(2):
                pl.semaphore_wait(sem.at[d], 1)

        def credit_signal(sem):
            for d in range(2):
                pl.semaphore_signal(
                    sem.at[d], inc=1,
                    device_id=(csrc[d],),
                    device_id_type=pl.DeviceIdType.MESH,
                )

        def pc(c, h, d):
            return jnp.dot(
                x_ref[pl.ds(c * CH, CH), :],
                w_ref[:, pl.ds(h * NH + d * NH2, NH2)],
                preferred_element_type=jnp.float32,
                precision=lax.Precision.HIGHEST,
            )

        def ring_start(buf_ref, s_slot, r_slot):
            rs = []
            for d in range(2):
                r = pltpu.make_async_remote_copy(
                    src_ref=buf_ref.at[d * 2 + s_slot],
                    dst_ref=buf_ref.at[d * 2 + r_slot],
                    send_sem=send_sem.at[d], recv_sem=recv_sem.at[d],
                    device_id=(dst[d],), device_id_type=pl.DeviceIdType.MESH,
                )
                r.start()
                rs.append(r)
            return rs

        def out_at(c, h, d):
            return o_ref.at[pl.ds(c * CH, CH), pl.ds(h * NH + d * NH2, NH2)]

        DQ = (slice(None), pl.ds(0, NH2))
        DQ1 = (slice(None), pl.ds(NH2, NH2))

        def deq_half(d):
            return deq_ref.at[DQ] if d == 0 else deq_ref.at[DQ1]

        def store_halves(c0, c1, h):
            cps = []
            for d, c in ((0, c0), (1, c1)):
                cp = pltpu.make_async_copy(
                    deq_half(d), out_at(c, h, d), store_sem.at[d]
                )
                cp.start()
                cps.append(cp)
            return cps

        def wait_prev_stores(h):
            for d in range(2):
                pltpu.make_async_copy(
                    deq_half(d), out_at(own[d], h, d), store_sem.at[d]
                ).wait()

        local_am = None
        for h in range(NSPLIT):
            comm_ref[0] = pc(me, h, 0)
            comm_ref[2] = pc(me, h, 1)

            def rs_step(s, _, h=h):
                s_slot = s % 2
                r_slot = (s + 1) % 2
                if h == 0:
                    @pl.when(s > 0)
                    def _():
                        credit_wait(rs_credit)
                else:
                    credit_wait(rs_credit)
                rdmas = ring_start(comm_ref, s_slot, r_slot)
                deq_ref[DQ] = pc((me - s - 1) % N_DEV, h, 0)
                deq_ref[DQ1] = pc((me + s + 1) % N_DEV, h, 1)
                for r in rdmas:
                    r.wait()
                comm_ref[0 + r_slot] = comm_ref[0 + r_slot] + deq_ref[DQ]
                comm_ref[2 + r_slot] = comm_ref[2 + r_slot] + deq_ref[DQ1]

                @pl.when(s < N_DEV - 2)
                def _():
                    credit_signal(rs_credit)
                return 0

            lax.fori_loop(0, N_DEV - 1, rs_step, 0)
            am_h = jnp.maximum(
                jnp.max(jnp.abs(comm_ref[1])), jnp.max(jnp.abs(comm_ref[3]))
            )
            local_am = am_h if local_am is None else jnp.maximum(local_am, am_h)
            parks = []
            for d, slot in ((0, 1), (1, 3)):
                cp = pltpu.make_async_copy(
                    comm_ref.at[slot], out_at(own[d], h, d), store_sem.at[d]
                )
                cp.start()
                parks.append(cp)
            for cp in parks:
                cp.wait()
            credit_signal(rs_credit if h == 0 else ag_credit)

        amx_ref[me] = jnp.broadcast_to(local_am, (8, 128))
        sends = []
        for k in range(1, N_DEV):
            j = (me + k) % N_DEV
            r = pltpu.make_async_remote_copy(
                src_ref=amx_ref.at[me], dst_ref=amx_ref.at[me],
                send_sem=amx_send.at[k - 1], recv_sem=amx_recv.at[me],
                device_id=(j,), device_id_type=pl.DeviceIdType.MESH,
            )
            r.start()
            sends.append(r)
        for k in range(1, N_DEV):
            j = (me + k) % N_DEV
            pltpu.make_async_remote_copy(
                src_ref=amx_ref.at[j], dst_ref=amx_ref.at[j],
                send_sem=amx_send.at[k - 1], recv_sem=amx_recv.at[j],
                device_id=(j,), device_id_type=pl.DeviceIdType.MESH,
            ).wait_recv()
        for r in sends:
            r.wait_send()
        g = amx_ref[0]
        for j in range(1, N_DEV):
            g = jnp.maximum(g, amx_ref[j])
        g_am = jnp.max(g)
        scale = g_am / 448.0
        inv = 448.0 / g_am

        for h in range(NSPLIT):
            if h > 0:
                wait_prev_stores(h - 1)
            loads = []
            for d in range(2):
                cp = pltpu.make_async_copy(
                    out_at(own[d], h, d), deq_half(d), load_sem.at[d]
                )
                cp.start()
                loads.append(cp)
            for cp in loads:
                cp.wait()
            q_ref[0] = (deq_ref[DQ] * inv).astype(jnp.float8_e4m3fn)
            q_ref[2] = (deq_ref[DQ1] * inv).astype(jnp.float8_e4m3fn)
            deq_ref[DQ] = q_ref[0].astype(jnp.float32) * scale
            deq_ref[DQ1] = q_ref[2].astype(jnp.float32) * scale
            for cp in store_halves(own[0], own[1], h):
                cp.wait()

            def ag_step(t, _, h=h):
                s_slot = t % 2
                r_slot = (t + 1) % 2
                credit_wait(ag_credit)
                rdmas = ring_start(q_ref, s_slot, r_slot)

                @pl.when(t > 1)
                def _():
                    wait_prev_stores(h)

                @pl.when(t > 0)
                def _():
                    deq_ref[DQ] = q_ref[0 + s_slot].astype(jnp.float32) * scale
                    deq_ref[DQ1] = q_ref[2 + s_slot].astype(jnp.float32) * scale
                    store_halves(
                        (me - (t - 1)) % N_DEV, (me + (t - 1)) % N_DEV, h
                    )
                for r in rdmas:
                    r.wait()
                @pl.when(t < N_DEV - 2)
                def _():
                    credit_signal(ag_credit)
                return 0

            lax.fori_loop(0, N_DEV - 1, ag_step, 0)
            wait_prev_stores(h)
            deq_ref[DQ] = q_ref[1].astype(jnp.float32) * scale
            deq_ref[DQ1] = q_ref[3].astype(jnp.float32) * scale
            store_halves(
                (me - (N_DEV - 2)) % N_DEV, (me + (N_DEV - 2)) % N_DEV, h
            )
            if h < NSPLIT - 1:
                credit_signal(ag_credit)
        wait_prev_stores(NSPLIT - 1)

    return pl.pallas_call(
        body,
        out_shape=jax.ShapeDtypeStruct((M, N), jnp.float32),
        in_specs=[
            pl.BlockSpec(memory_space=pltpu.MemorySpace.VMEM),
            pl.BlockSpec(memory_space=pltpu.MemorySpace.VMEM),
        ],
        out_specs=pl.BlockSpec(memory_space=pl.ANY),
        scratch_shapes=[
            pltpu.VMEM((4, CH, NH2), jnp.float32),
            pltpu.VMEM((4, CH, NH2), jnp.float8_e4m3fn),
            pltpu.VMEM((CH, NH), jnp.float32),
            pltpu.VMEM((N_DEV, 8, 128), jnp.float32),
            pltpu.SemaphoreType.DMA((2,)),
            pltpu.SemaphoreType.DMA((2,)),
            pltpu.SemaphoreType.DMA((N_DEV - 1,)),
            pltpu.SemaphoreType.DMA((N_DEV,)),
            pltpu.SemaphoreType.DMA((2,)),
            pltpu.SemaphoreType.DMA((2,)),
            pltpu.SemaphoreType.REGULAR((2,)),
            pltpu.SemaphoreType.REGULAR((2,)),
        ],
        compiler_params=pltpu.CompilerParams(
            collective_id=0, vmem_limit_bytes=63 * 1024 * 1024
        ),
    )(x, w)


def kernel(x, w_mat):
    return _fused(x, w_mat)
